# baseline (speedup 1.0000x reference)
import jax
import jax.numpy as jnp
import numpy as np
from jax.experimental import pallas as pl
from jax.experimental.pallas import tpu as pltpu

EMBED_DIM = 128
NUM_HEADS = 8
HEAD_DIM = EMBED_DIM // NUM_HEADS
SEQ = 32
NTOK = 750
TBLK = 75
ROWS = TBLK * SEQ
HS = NUM_HEADS * SEQ


def _consts():
    half = HEAD_DIM // 2
    angle = 1.0 / 10000.0 ** np.linspace(0.0, 1.0, half)
    angle = np.repeat(angle, 2)
    angle_full = np.tile(angle, NUM_HEADS)
    idx = np.arange(SEQ, dtype=np.float64)
    sin = np.sin(idx[:, None] * angle_full[None, :])
    cos = np.cos(idx[:, None] * angle_full[None, :])

    P16 = np.zeros((HEAD_DIM, HEAD_DIM), np.float32)
    for o in range(half):
        P16[2 * o + 1, o] = -1.0
    for o in range(half, HEAD_DIM):
        P16[2 * (o - half), o] = 1.0
    P = np.zeros((EMBED_DIM, EMBED_DIM), np.float32)
    for h in range(NUM_HEADS):
        P[h * 16:(h + 1) * 16, h * 16:(h + 1) * 16] = P16

    decay = np.log(1.0 - 2.0 ** (-1.0 - 3.0 * np.arange(NUM_HEADS, dtype=np.float64) / NUM_HEADS))
    ij = np.abs(idx[:, None] - idx[None, :])
    mask3 = np.transpose(decay[:, None, None] * ij[None], (1, 0, 2)).reshape(SEQ, HS)

    fm = np.zeros((NUM_HEADS, EMBED_DIM), np.float32)
    for h in range(NUM_HEADS):
        fm[h, h * 16:(h + 1) * 16] = 1.0
    MS = np.repeat(fm, SEQ, axis=0)
    return (cos.astype(np.float32), sin.astype(np.float32), P,
            mask3.astype(np.float32), fm, MS)


_COS, _SIN, _P, _MASK3, _FM, _MS = _consts()


def _ln(x, g, b, eps=1e-5):
    mu = jnp.mean(x, axis=-1, keepdims=True)
    var = jnp.mean((x - mu) ** 2, axis=-1, keepdims=True)
    return (x - mu) * jax.lax.rsqrt(var + eps) * g + b


def _block_kernel(xp_ref, wq_ref, bq_ref, wk_ref, bk_ref, wv_ref, bv_ref,
                  g1_ref, be1_ref, w1_ref, b1_ref, w2_ref, b2_ref,
                  g2_ref, be2_ref, cos_ref, sin_ref, p_ref, mask_ref,
                  fm_ref, ms_ref, out_ref,
                  qs_ref, ks_ref, vs_ref, es_ref, os_ref):
    f32 = jnp.float32
    bf16 = jnp.bfloat16
    xb = xp_ref[:].reshape(ROWS, EMBED_DIM)
    xb_bf = xb.astype(bf16)

    def mm(a, b, prefer=f32):
        return jax.lax.dot_general(a, b, (((1,), (0,)), ((), ())),
                                   preferred_element_type=prefer)

    def mm_nt(a, b, prefer=f32):
        return jax.lax.dot_general(a, b, (((1,), (1,)), ((), ())),
                                   preferred_element_type=prefer)

    cos = cos_ref[:]
    sin = sin_ref[:]
    P = p_ref[:]
    fm = fm_ref[:]

    def rot_bf(t_bf):
        tp = mm(t_bf, P).astype(bf16)
        t3 = t_bf.reshape(SEQ, TBLK, EMBED_DIM)
        tp3 = tp.reshape(SEQ, TBLK, EMBED_DIM)
        return t3 * cos[:, None, :] + tp3 * sin[:, None, :]

    q_bf = (mm(xb_bf, wq_ref[:]) + bq_ref[:]).astype(bf16)
    k_bf = (mm(xb_bf, wk_ref[:]) + bk_ref[:]).astype(bf16)
    v_bf = (mm(xb_bf, wv_ref[:]) + bv_ref[:]).astype(bf16)

    qs_ref[:] = rot_bf(q_bf)
    ks_ref[:] = rot_bf(k_bf)
    vs_ref[:] = v_bf.reshape(SEQ, TBLK, EMBED_DIM)

    mask3 = mask_ref[:]

    for t in range(TBLK):
        q_t = qs_ref[:, t, :]
        k_t = ks_ref[:, t, :]
        v_t = vs_ref[:, t, :]
        km = (k_t[None] * fm[:, None, :]).reshape(HS, EMBED_DIM)
        vm = (v_t[None] * fm[:, None, :]).reshape(HS, EMBED_DIM)
        s3 = mm_nt(q_t, km)
        e_t = jnp.exp(s3 + mask3).astype(bf16)
        es_ref[:, t, :] = e_t
        os_ref[:, t, :] = mm(e_t, vm)

    e_all = es_ref[:].reshape(ROWS, HS)
    den = mm(e_all, ms_ref[:])
    attn = os_ref[:].reshape(ROWS, EMBED_DIM) / den

    y = _ln(attn + xb, g1_ref[:], be1_ref[:])
    h1 = jnp.maximum(mm(y.astype(bf16), w1_ref[:]) + b1_ref[:], 0.0)
    ffn = mm(h1.astype(bf16), w2_ref[:]) + b2_ref[:]
    out_ref[:] = _ln(ffn + y, g2_ref[:], be2_ref[:]).reshape(SEQ, 1, TBLK, EMBED_DIM)


@jax.jit
def kernel(x, Wq, bq, Wk, bk, Wv, bv, g1, be1, W1, b1, W2, b2, g2, be2):
    B, N, C = x.shape
    bf16 = jnp.bfloat16

    grid = N // TBLK
    full = lambda shape: pl.BlockSpec(shape, lambda i: (0,) * len(shape))
    out = pl.pallas_call(
        _block_kernel,
        grid=(grid,),
        in_specs=[
            pl.BlockSpec((SEQ, 1, TBLK, C), lambda i: (0, i, 0, 0)),
            full((C, C)), full((1, C)),
            full((C, C)), full((1, C)),
            full((C, C)), full((1, C)),
            full((1, C)), full((1, C)),
            full((C, 4 * C)), full((1, 4 * C)),
            full((4 * C, C)), full((1, C)),
            full((1, C)), full((1, C)),
            full((SEQ, C)), full((SEQ, C)), full((C, C)),
            full((SEQ, HS)), full((NUM_HEADS, C)), full((HS, C)),
        ],
        out_specs=pl.BlockSpec((SEQ, 1, TBLK, C), lambda i: (0, i, 0, 0)),
        out_shape=jax.ShapeDtypeStruct((B, grid, TBLK, C), jnp.float32),
        scratch_shapes=[
            pltpu.VMEM((SEQ, TBLK, C), bf16),
            pltpu.VMEM((SEQ, TBLK, C), bf16),
            pltpu.VMEM((SEQ, TBLK, C), bf16),
            pltpu.VMEM((SEQ, TBLK, HS), bf16),
            pltpu.VMEM((SEQ, TBLK, C), jnp.float32),
        ],
        compiler_params=pltpu.CompilerParams(
            dimension_semantics=("parallel",)),
    )(x.reshape(B, grid, TBLK, C), Wq.astype(bf16), bq.reshape(1, C),
      Wk.astype(bf16), bk.reshape(1, C), Wv.astype(bf16), bv.reshape(1, C),
      g1.reshape(1, C), be1.reshape(1, C),
      W1.astype(bf16), b1.reshape(1, 4 * C),
      W2.astype(bf16), b2.reshape(1, C), g2.reshape(1, C), be2.reshape(1, C),
      jnp.asarray(_COS, bf16), jnp.asarray(_SIN, bf16),
      jnp.asarray(_P, bf16), jnp.asarray(_MASK3),
      jnp.asarray(_FM, bf16), jnp.asarray(_MS, bf16))

    return out.reshape(B, N, C)


# speedup vs baseline: 10.4836x; 10.4836x over previous
import jax
import jax.numpy as jnp
import numpy as np
from jax.experimental import pallas as pl
from jax.experimental.pallas import tpu as pltpu

EMBED_DIM = 128
NUM_HEADS = 8
HEAD_DIM = EMBED_DIM // NUM_HEADS
SEQ = 32
NTOK = 750
TBLK = 125
ROWS = TBLK * SEQ
HS = NUM_HEADS * SEQ


def _consts():
    half = HEAD_DIM // 2
    angle = 1.0 / 10000.0 ** np.linspace(0.0, 1.0, half)
    angle = np.repeat(angle, 2)
    angle_full = np.tile(angle, NUM_HEADS)
    idx = np.arange(SEQ, dtype=np.float64)
    sin = np.sin(idx[:, None] * angle_full[None, :])
    cos = np.cos(idx[:, None] * angle_full[None, :])

    P16 = np.zeros((HEAD_DIM, HEAD_DIM), np.float32)
    for o in range(half):
        P16[2 * o + 1, o] = -1.0
    for o in range(half, HEAD_DIM):
        P16[2 * (o - half), o] = 1.0
    P = np.zeros((EMBED_DIM, EMBED_DIM), np.float32)
    for h in range(NUM_HEADS):
        P[h * 16:(h + 1) * 16, h * 16:(h + 1) * 16] = P16

    decay = np.log(1.0 - 2.0 ** (-1.0 - 3.0 * np.arange(NUM_HEADS, dtype=np.float64) / NUM_HEADS))
    ij = np.abs(idx[:, None] - idx[None, :])
    mask3 = np.transpose(decay[:, None, None] * ij[None], (1, 0, 2)).reshape(SEQ, HS)

    fm = np.zeros((NUM_HEADS, EMBED_DIM), np.float32)
    for h in range(NUM_HEADS):
        fm[h, h * 16:(h + 1) * 16] = 1.0
    MS = np.repeat(fm, SEQ, axis=0)
    return (cos.astype(np.float32), sin.astype(np.float32), P,
            mask3.astype(np.float32), fm, MS)


_COS, _SIN, _P, _MASK3, _FM, _MS = _consts()


def _ln(x, g, b, eps=1e-5):
    mu = jnp.mean(x, axis=-1, keepdims=True)
    var = jnp.mean((x - mu) ** 2, axis=-1, keepdims=True)
    return (x - mu) * jax.lax.rsqrt(var + eps) * g + b


def _block_kernel(xp_ref, wq_ref, bq_ref, wk_ref, bk_ref, wv_ref, bv_ref,
                  g1_ref, be1_ref, w1_ref, b1_ref, w2_ref, b2_ref,
                  g2_ref, be2_ref, cos_ref, sin_ref, p_ref, mask_ref,
                  fm_ref, ms_ref, out_ref):
    f32 = jnp.float32
    bf16 = jnp.bfloat16
    xb = xp_ref[:]
    xb_bf = xb.astype(bf16)

    def mm(a, b, prefer=f32):
        return jax.lax.dot_general(a, b, (((1,), (0,)), ((), ())),
                                   preferred_element_type=prefer)

    def mm_nt(a, b, prefer=f32):
        return jax.lax.dot_general(a, b, (((1,), (1,)), ((), ())),
                                   preferred_element_type=prefer)

    cos = cos_ref[:]
    sin = sin_ref[:]
    P = p_ref[:]
    fm = fm_ref[:]

    def rot_bf(t_bf):
        tp = mm(t_bf, P).astype(bf16)
        t3 = t_bf.reshape(TBLK, SEQ, EMBED_DIM)
        tp3 = tp.reshape(TBLK, SEQ, EMBED_DIM)
        return (t3 * cos[None] + tp3 * sin[None]).reshape(ROWS, EMBED_DIM)

    q_bf = (mm(xb_bf, wq_ref[:]) + bq_ref[:]).astype(bf16)
    k_bf = (mm(xb_bf, wk_ref[:]) + bk_ref[:]).astype(bf16)
    v_bf = (mm(xb_bf, wv_ref[:]) + bv_ref[:]).astype(bf16)

    qr = rot_bf(q_bf)
    kr = rot_bf(k_bf)

    km = (kr.reshape(TBLK, 1, SEQ, EMBED_DIM) * fm[None, :, None, :]
          ).reshape(TBLK * HS, EMBED_DIM)
    vm = (v_bf.reshape(TBLK, 1, SEQ, EMBED_DIM) * fm[None, :, None, :]
          ).reshape(TBLK * HS, EMBED_DIM)

    mask3 = mask_ref[:]

    e_list = []
    for t in range(TBLK):
        s3 = mm_nt(qr[t * SEQ:(t + 1) * SEQ], km[t * HS:(t + 1) * HS])
        e_list.append(jnp.exp(s3 + mask3).astype(bf16))
    e_all = jnp.concatenate(e_list, axis=0)

    den = mm(e_all, ms_ref[:])

    o_list = []
    for t in range(TBLK):
        onum = mm(e_list[t], vm[t * HS:(t + 1) * HS])
        o_list.append(onum)
    attn = jnp.concatenate(o_list, axis=0) / den

    y = _ln(attn + xb, g1_ref[:], be1_ref[:])
    h1 = jnp.maximum(mm(y.astype(bf16), w1_ref[:]) + b1_ref[:], 0.0)
    ffn = mm(h1.astype(bf16), w2_ref[:]) + b2_ref[:]
    out_ref[:] = _ln(ffn + y, g2_ref[:], be2_ref[:])


@jax.jit
def kernel(x, Wq, bq, Wk, bk, Wv, bv, g1, be1, W1, b1, W2, b2, g2, be2):
    B, N, C = x.shape
    xp = jnp.transpose(x, (1, 0, 2)).reshape(N * B, C)
    bf16 = jnp.bfloat16

    grid = N // TBLK
    full = lambda shape: pl.BlockSpec(shape, lambda i: (0,) * len(shape))
    out = pl.pallas_call(
        _block_kernel,
        grid=(grid,),
        in_specs=[
            pl.BlockSpec((ROWS, C), lambda i: (i, 0)),
            full((C, C)), full((1, C)),
            full((C, C)), full((1, C)),
            full((C, C)), full((1, C)),
            full((1, C)), full((1, C)),
            full((C, 4 * C)), full((1, 4 * C)),
            full((4 * C, C)), full((1, C)),
            full((1, C)), full((1, C)),
            full((SEQ, C)), full((SEQ, C)), full((C, C)),
            full((SEQ, HS)), full((NUM_HEADS, C)), full((HS, C)),
        ],
        out_specs=pl.BlockSpec((ROWS, C), lambda i: (i, 0)),
        out_shape=jax.ShapeDtypeStruct((N * B, C), jnp.float32),
        compiler_params=pltpu.CompilerParams(
            dimension_semantics=("parallel",)),
    )(xp, Wq.astype(bf16), bq.reshape(1, C), Wk.astype(bf16),
      bk.reshape(1, C), Wv.astype(bf16), bv.reshape(1, C),
      g1.reshape(1, C), be1.reshape(1, C),
      W1.astype(bf16), b1.reshape(1, 4 * C),
      W2.astype(bf16), b2.reshape(1, C), g2.reshape(1, C), be2.reshape(1, C),
      jnp.asarray(_COS, bf16), jnp.asarray(_SIN, bf16),
      jnp.asarray(_P, bf16), jnp.asarray(_MASK3),
      jnp.asarray(_FM, bf16), jnp.asarray(_MS, bf16))

    return out.reshape(N, B, C).transpose(1, 0, 2)


# speedup vs baseline: 10.7132x; 1.0219x over previous
import jax
import jax.numpy as jnp
import numpy as np
from jax.experimental import pallas as pl
from jax.experimental.pallas import tpu as pltpu

EMBED_DIM = 128
NUM_HEADS = 8
HEAD_DIM = EMBED_DIM // NUM_HEADS
SEQ = 32
NTOK = 750
TBLK = 150
ROWS = TBLK * SEQ
HS = NUM_HEADS * SEQ


def _consts():
    half = HEAD_DIM // 2
    angle = 1.0 / 10000.0 ** np.linspace(0.0, 1.0, half)
    angle = np.repeat(angle, 2)
    angle_full = np.tile(angle, NUM_HEADS)
    idx = np.arange(SEQ, dtype=np.float64)
    sin = np.sin(idx[:, None] * angle_full[None, :])
    cos = np.cos(idx[:, None] * angle_full[None, :])

    P16 = np.zeros((HEAD_DIM, HEAD_DIM), np.float32)
    for o in range(half):
        P16[2 * o + 1, o] = -1.0
    for o in range(half, HEAD_DIM):
        P16[2 * (o - half), o] = 1.0
    P = np.zeros((EMBED_DIM, EMBED_DIM), np.float32)
    for h in range(NUM_HEADS):
        P[h * 16:(h + 1) * 16, h * 16:(h + 1) * 16] = P16

    decay = np.log(1.0 - 2.0 ** (-1.0 - 3.0 * np.arange(NUM_HEADS, dtype=np.float64) / NUM_HEADS))
    ij = np.abs(idx[:, None] - idx[None, :])
    mask3 = np.transpose(decay[:, None, None] * ij[None], (1, 0, 2)).reshape(SEQ, HS)

    fm = np.zeros((NUM_HEADS, EMBED_DIM), np.float32)
    for h in range(NUM_HEADS):
        fm[h, h * 16:(h + 1) * 16] = 1.0
    MS = np.repeat(fm, SEQ, axis=0)
    return (cos.astype(np.float32), sin.astype(np.float32), P,
            mask3.astype(np.float32), fm, MS)


_COS, _SIN, _P, _MASK3, _FM, _MS = _consts()


def _ln(x, g, b, eps=1e-5):
    mu = jnp.mean(x, axis=-1, keepdims=True)
    var = jnp.mean((x - mu) ** 2, axis=-1, keepdims=True)
    return (x - mu) * jax.lax.rsqrt(var + eps) * g + b


def _block_kernel(xp_ref, wq_ref, bq_ref, wk_ref, bk_ref, wv_ref, bv_ref,
                  g1_ref, be1_ref, w1_ref, b1_ref, w2_ref, b2_ref,
                  g2_ref, be2_ref, cos_ref, sin_ref, p_ref, mask_ref,
                  fm_ref, ms_ref, out_ref):
    f32 = jnp.float32
    bf16 = jnp.bfloat16
    xb = xp_ref[:]
    xb_bf = xb.astype(bf16)

    def mm(a, b, prefer=f32):
        return jax.lax.dot_general(a, b, (((1,), (0,)), ((), ())),
                                   preferred_element_type=prefer)

    def mm_nt(a, b, prefer=f32):
        return jax.lax.dot_general(a, b, (((1,), (1,)), ((), ())),
                                   preferred_element_type=prefer)

    cos = cos_ref[:]
    sin = sin_ref[:]
    P = p_ref[:]
    fm = fm_ref[:]

    def rot_bf(t_bf):
        tp = mm(t_bf, P).astype(bf16)
        t3 = t_bf.reshape(TBLK, SEQ, EMBED_DIM)
        tp3 = tp.reshape(TBLK, SEQ, EMBED_DIM)
        return (t3 * cos[None] + tp3 * sin[None]).reshape(ROWS, EMBED_DIM)

    q_bf = (mm(xb_bf, wq_ref[:]) + bq_ref[:]).astype(bf16)
    k_bf = (mm(xb_bf, wk_ref[:]) + bk_ref[:]).astype(bf16)
    v_bf = (mm(xb_bf, wv_ref[:]) + bv_ref[:]).astype(bf16)

    qr = rot_bf(q_bf)
    kr = rot_bf(k_bf)

    km = (kr.reshape(TBLK, 1, SEQ, EMBED_DIM) * fm[None, :, None, :]
          ).reshape(TBLK * HS, EMBED_DIM)
    vm = (v_bf.reshape(TBLK, 1, SEQ, EMBED_DIM) * fm[None, :, None, :]
          ).reshape(TBLK * HS, EMBED_DIM)

    mask3 = mask_ref[:]

    e_list = []
    for t in range(TBLK):
        s3 = mm_nt(qr[t * SEQ:(t + 1) * SEQ], km[t * HS:(t + 1) * HS])
        e_list.append(jnp.exp(s3 + mask3).astype(bf16))
    e_all = jnp.concatenate(e_list, axis=0)

    den = mm(e_all, ms_ref[:])

    o_list = []
    for t in range(TBLK):
        onum = mm(e_list[t], vm[t * HS:(t + 1) * HS])
        o_list.append(onum)
    attn = jnp.concatenate(o_list, axis=0) / den

    y = _ln(attn + xb, g1_ref[:], be1_ref[:])
    h1 = jnp.maximum(mm(y.astype(bf16), w1_ref[:]) + b1_ref[:], 0.0)
    ffn = mm(h1.astype(bf16), w2_ref[:]) + b2_ref[:]
    out_ref[:] = _ln(ffn + y, g2_ref[:], be2_ref[:])


@jax.jit
def kernel(x, Wq, bq, Wk, bk, Wv, bv, g1, be1, W1, b1, W2, b2, g2, be2):
    B, N, C = x.shape
    xp = jnp.transpose(x, (1, 0, 2)).reshape(N * B, C)
    bf16 = jnp.bfloat16

    grid = N // TBLK
    full = lambda shape: pl.BlockSpec(shape, lambda i: (0,) * len(shape))
    out = pl.pallas_call(
        _block_kernel,
        grid=(grid,),
        in_specs=[
            pl.BlockSpec((ROWS, C), lambda i: (i, 0)),
            full((C, C)), full((1, C)),
            full((C, C)), full((1, C)),
            full((C, C)), full((1, C)),
            full((1, C)), full((1, C)),
            full((C, 4 * C)), full((1, 4 * C)),
            full((4 * C, C)), full((1, C)),
            full((1, C)), full((1, C)),
            full((SEQ, C)), full((SEQ, C)), full((C, C)),
            full((SEQ, HS)), full((NUM_HEADS, C)), full((HS, C)),
        ],
        out_specs=pl.BlockSpec((ROWS, C), lambda i: (i, 0)),
        out_shape=jax.ShapeDtypeStruct((N * B, C), jnp.float32),
        compiler_params=pltpu.CompilerParams(
            dimension_semantics=("parallel",)),
    )(xp, Wq.astype(bf16), bq.reshape(1, C), Wk.astype(bf16),
      bk.reshape(1, C), Wv.astype(bf16), bv.reshape(1, C),
      g1.reshape(1, C), be1.reshape(1, C),
      W1.astype(bf16), b1.reshape(1, 4 * C),
      W2.astype(bf16), b2.reshape(1, C), g2.reshape(1, C), be2.reshape(1, C),
      jnp.asarray(_COS, bf16), jnp.asarray(_SIN, bf16),
      jnp.asarray(_P, bf16), jnp.asarray(_MASK3),
      jnp.asarray(_FM, bf16), jnp.asarray(_MS, bf16))

    return out.reshape(N, B, C).transpose(1, 0, 2)


# speedup vs baseline: 11.5241x; 1.0757x over previous
import jax
import jax.numpy as jnp
import numpy as np
from jax.experimental import pallas as pl
from jax.experimental.pallas import tpu as pltpu

EMBED_DIM = 128
NUM_HEADS = 8
HEAD_DIM = EMBED_DIM // NUM_HEADS
SEQ = 32
NTOK = 750
TBLK = 150
ROWS = TBLK * SEQ
HS = NUM_HEADS * SEQ


def _consts():
    half = HEAD_DIM // 2
    angle = 1.0 / 10000.0 ** np.linspace(0.0, 1.0, half)
    angle = np.repeat(angle, 2)
    angle_full = np.tile(angle, NUM_HEADS)
    idx = np.arange(SEQ, dtype=np.float64)
    sin = np.sin(idx[:, None] * angle_full[None, :])
    cos = np.cos(idx[:, None] * angle_full[None, :])

    P16 = np.zeros((HEAD_DIM, HEAD_DIM), np.float32)
    for o in range(half):
        P16[2 * o + 1, o] = -1.0
    for o in range(half, HEAD_DIM):
        P16[2 * (o - half), o] = 1.0
    P = np.zeros((EMBED_DIM, EMBED_DIM), np.float32)
    for h in range(NUM_HEADS):
        P[h * 16:(h + 1) * 16, h * 16:(h + 1) * 16] = P16

    decay = np.log(1.0 - 2.0 ** (-1.0 - 3.0 * np.arange(NUM_HEADS, dtype=np.float64) / NUM_HEADS))
    ij = np.abs(idx[:, None] - idx[None, :])
    mask3 = np.transpose(decay[:, None, None] * ij[None], (1, 0, 2)).reshape(SEQ, HS)

    fm = np.zeros((NUM_HEADS, EMBED_DIM), np.float32)
    for h in range(NUM_HEADS):
        fm[h, h * 16:(h + 1) * 16] = 1.0
    MS = np.repeat(fm, SEQ, axis=0)
    gidx = np.zeros((EMBED_DIM,), np.int32)
    sgn = np.ones((EMBED_DIM,), np.float64)
    for g in range(NUM_HEADS):
        for o in range(HEAD_DIM):
            if o < half:
                gidx[16 * g + o] = 16 * g + 2 * o + 1
                sgn[16 * g + o] = -1.0
            else:
                gidx[16 * g + o] = 16 * g + 2 * (o - half)
    sin = sin * sgn[None, :]
    return (cos.astype(np.float32), sin.astype(np.float32), P,
            mask3.astype(np.float32), fm, MS, gidx)


_COS, _SIN, _P, _MASK3, _FM, _MS, _GIDX = _consts()


def _ln(x, g, b, eps=1e-5):
    mu = jnp.mean(x, axis=-1, keepdims=True)
    var = jnp.mean((x - mu) ** 2, axis=-1, keepdims=True)
    return (x - mu) * jax.lax.rsqrt(var + eps) * g + b


def _block_kernel(xp_ref, wq_ref, bq_ref, wk_ref, bk_ref, wv_ref, bv_ref,
                  g1_ref, be1_ref, w1_ref, b1_ref, w2_ref, b2_ref,
                  g2_ref, be2_ref, cos_ref, sin_ref, p_ref, mask_ref,
                  fm_ref, ms_ref, gidx_ref, out_ref):
    f32 = jnp.float32
    bf16 = jnp.bfloat16
    xb = xp_ref[:]
    xb_bf = xb.astype(bf16)

    def mm(a, b, prefer=f32):
        return jax.lax.dot_general(a, b, (((1,), (0,)), ((), ())),
                                   preferred_element_type=prefer)

    def mm_nt(a, b, prefer=f32):
        return jax.lax.dot_general(a, b, (((1,), (1,)), ((), ())),
                                   preferred_element_type=prefer)

    cos = cos_ref[:]
    sin = sin_ref[:]
    P = p_ref[:]
    fm = fm_ref[:]

    gidx = gidx_ref[:]
    def rot_bf(t_f):
        tp = jnp.take_along_axis(t_f, jnp.broadcast_to(gidx, t_f.shape), axis=1)
        t3 = t_f.reshape(TBLK, SEQ, EMBED_DIM)
        tp3 = tp.reshape(TBLK, SEQ, EMBED_DIM)
        return (t3 * cos[None] + tp3 * sin[None]).reshape(ROWS, EMBED_DIM).astype(bf16)

    q_f = mm(xb_bf, wq_ref[:]) + bq_ref[:]
    k_f = mm(xb_bf, wk_ref[:]) + bk_ref[:]
    v_bf = (mm(xb_bf, wv_ref[:]) + bv_ref[:]).astype(bf16)

    qr = rot_bf(q_f)
    kr = rot_bf(k_f)

    km = (kr.reshape(TBLK, 1, SEQ, EMBED_DIM) * fm[None, :, None, :]
          ).reshape(TBLK * HS, EMBED_DIM)
    vm = (v_bf.reshape(TBLK, 1, SEQ, EMBED_DIM) * fm[None, :, None, :]
          ).reshape(TBLK * HS, EMBED_DIM)

    mask3 = mask_ref[:]

    e_list = []
    for t in range(TBLK):
        s3 = mm_nt(qr[t * SEQ:(t + 1) * SEQ], km[t * HS:(t + 1) * HS])
        e_list.append(jnp.exp(s3 + mask3).astype(bf16))
    e_all = jnp.concatenate(e_list, axis=0)

    den = mm(e_all, ms_ref[:])

    o_list = []
    for t in range(TBLK):
        onum = mm(e_list[t], vm[t * HS:(t + 1) * HS])
        o_list.append(onum)
    attn = jnp.concatenate(o_list, axis=0) / den

    y = _ln(attn + xb, g1_ref[:], be1_ref[:])
    h1 = jnp.maximum(mm(y.astype(bf16), w1_ref[:]) + b1_ref[:], 0.0)
    ffn = mm(h1.astype(bf16), w2_ref[:]) + b2_ref[:]
    out_ref[:] = _ln(ffn + y, g2_ref[:], be2_ref[:])


@jax.jit
def kernel(x, Wq, bq, Wk, bk, Wv, bv, g1, be1, W1, b1, W2, b2, g2, be2):
    B, N, C = x.shape
    xp = jnp.transpose(x, (1, 0, 2)).reshape(N * B, C)
    bf16 = jnp.bfloat16

    grid = N // TBLK
    full = lambda shape: pl.BlockSpec(shape, lambda i: (0,) * len(shape))
    out = pl.pallas_call(
        _block_kernel,
        grid=(grid,),
        in_specs=[
            pl.BlockSpec((ROWS, C), lambda i: (i, 0)),
            full((C, C)), full((1, C)),
            full((C, C)), full((1, C)),
            full((C, C)), full((1, C)),
            full((1, C)), full((1, C)),
            full((C, 4 * C)), full((1, 4 * C)),
            full((4 * C, C)), full((1, C)),
            full((1, C)), full((1, C)),
            full((SEQ, C)), full((SEQ, C)), full((C, C)),
            full((SEQ, HS)), full((NUM_HEADS, C)), full((HS, C)),
            full((1, C)),
        ],
        out_specs=pl.BlockSpec((ROWS, C), lambda i: (i, 0)),
        out_shape=jax.ShapeDtypeStruct((N * B, C), jnp.float32),
        compiler_params=pltpu.CompilerParams(
            dimension_semantics=("parallel",)),
    )(xp, Wq.astype(bf16), bq.reshape(1, C), Wk.astype(bf16),
      bk.reshape(1, C), Wv.astype(bf16), bv.reshape(1, C),
      g1.reshape(1, C), be1.reshape(1, C),
      W1.astype(bf16), b1.reshape(1, 4 * C),
      W2.astype(bf16), b2.reshape(1, C), g2.reshape(1, C), be2.reshape(1, C),
      jnp.asarray(_COS), jnp.asarray(_SIN),
      jnp.asarray(_P, bf16), jnp.asarray(_MASK3),
      jnp.asarray(_FM, bf16), jnp.asarray(_MS, bf16),
      jnp.asarray(_GIDX.reshape(1, C)))

    return out.reshape(N, B, C).transpose(1, 0, 2)
